# baseline (speedup 1.0000x reference)
import functools

import jax
import jax.numpy as jnp
from jax import lax
from jax.experimental import pallas as pl
from jax.experimental.pallas import tpu as pltpu
from jax.experimental.pallas import tpu_sc as plsc

_VOCAB = 1000000
_EMB = 64
_BATCH = 4096
_HIST = 200
_NW = 32
_NVC = _VOCAB // 128
_UNITS = _HIST * (_BATCH // 128)
_UPW = _UNITS // _NW

_mesh = plsc.VectorSubcoreMesh(core_axis_name="c", subcore_axis_name="s")


def _iota16():
    return lax.iota(jnp.int32, 16)


@functools.partial(
    pl.kernel,
    mesh=_mesh,
    out_type=jax.ShapeDtypeStruct((_VOCAB * _EMB // 128, 128), jnp.float32),
    scratch_types=[
        pltpu.VMEM((64, 128), jnp.float32),
        pltpu.VMEM((64, 128), jnp.float32),
    ],
    compiler_params=pltpu.CompilerParams(
        use_tc_tiling_on_sc=True, needs_layout_passes=False),
)
def _detile(tT_hbm, tail_hbm, rowtab_hbm, buf_v, tr_v):
    wid = lax.axis_index("s") * 2 + lax.axis_index("c")
    iota = _iota16()

    def transpose_slab(n_vl):
        def col_group(g, c):
            for sub in range(4):
                vl = g * 4 + sub
                colv = jnp.full((16,), 0, jnp.int32) + vl
                dst_row = vl // 2
                dst_off = 64 * (vl % 2)
                for h in range(4):
                    vec = plsc.load_gather(buf_v, [iota + 16 * h, colv])
                    tr_v[dst_row, pl.ds(dst_off + 16 * h, 16)] = vec
            return c

        lax.fori_loop(0, n_vl // 4, col_group, 0)

    def slab(vc, carry):
        @pl.when(vc < _NVC)
        def _():
            pltpu.sync_copy(tT_hbm.at[:, pl.ds(vc * 128, 128)], buf_v)
            transpose_slab(128)
            pltpu.sync_copy(tr_v, rowtab_hbm.at[pl.ds(vc * 64, 64), :])

        return carry

    lax.fori_loop(0, (_NVC + _NW - 1) // _NW, lambda k, c: slab(wid + 32 * k, c), 0)

    @pl.when(wid == 4)
    def _tail():
        pltpu.sync_copy(tail_hbm, tr_v.at[pl.ds(0, 32), :])
        pltpu.sync_copy(tr_v.at[pl.ds(0, 32), :],
                        rowtab_hbm.at[pl.ds(_NVC * 64, 32), :])


@functools.partial(
    pl.kernel,
    mesh=_mesh,
    out_type=jax.ShapeDtypeStruct((_HIST, 8, _BATCH // 128, 8, 128), jnp.float32),
    scratch_types=[
        pltpu.VMEM((128,), jnp.int32),
        pltpu.VMEM((128, 64), jnp.float32),
        pltpu.VMEM((8, 8, 128), jnp.float32),
        pltpu.SemaphoreType.DMA,
    ],
    compiler_params=pltpu.CompilerParams(
        use_tc_tiling_on_sc=False, needs_layout_passes=False),
)
def _gather(xT_hbm, rowtab_hbm, out_hbm, idx_v, rows_v, tr_v, sem):
    wid = lax.axis_index("s") * 2 + lax.axis_index("c")
    iota = _iota16()

    def unit(u, carry):
        t = u // 32
        bc = lax.rem(u, 32)
        pltpu.sync_copy(xT_hbm.at[t, pl.ds(bc * 128, 128)], idx_v)
        pltpu.async_copy(rowtab_hbm.at[idx_v], rows_v, sem).wait()

        def row_group(h, c):
            base = 16 * h
            for e in range(_EMB):
                colv = jnp.full((16,), 0, jnp.int32) + e
                vec = plsc.load_gather(rows_v, [iota + base, colv])
                tr_v[e // 8, e % 8, pl.ds(base, 16)] = vec
            return c

        lax.fori_loop(0, 8, row_group, 0)
        pltpu.sync_copy(tr_v, out_hbm.at[t, :, bc])
        return carry

    lax.fori_loop(_UPW * wid, _UPW * (wid + 1), unit, 0)


def kernel(x, emb_table):
    tT = jnp.swapaxes(emb_table, 0, 1)
    xT = jnp.swapaxes(x, 0, 1).astype(jnp.int32)
    tail = emb_table[_NVC * 128:, :].reshape(32, 128)
    rowtab = _detile(tT, tail).reshape(_VOCAB, _EMB)
    out5 = _gather(xT, rowtab)
    return out5.transpose(2, 4, 0, 1, 3).reshape(_BATCH, _HIST, _EMB)


# speedup vs baseline: 1.5874x; 1.5874x over previous
import functools

import jax
import jax.numpy as jnp
from jax import lax
from jax.experimental import pallas as pl
from jax.experimental.pallas import tpu as pltpu
from jax.experimental.pallas import tpu_sc as plsc

_VOCAB = 1000000
_EMB = 64
_W = 80
_BATCH = 4096
_HIST = 200
_NW = 32
_NVC = _VOCAB // 128
_ROWS = _VOCAB * _W // 128
_TAILR = 64 * _W // 128
_UNITS = _HIST * (_BATCH // 128)
_UPW = _UNITS // _NW

_mesh = plsc.VectorSubcoreMesh(core_axis_name="c", subcore_axis_name="s")


@functools.partial(
    pl.kernel,
    mesh=_mesh,
    out_type=jax.ShapeDtypeStruct((_ROWS, 128), jnp.float32),
    scratch_types=[
        pltpu.VMEM((64, 128), jnp.float32),
        pltpu.VMEM((_W, 128), jnp.float32),
    ],
    compiler_params=pltpu.CompilerParams(
        use_tc_tiling_on_sc=True, needs_layout_passes=False),
)
def _detile(tT_hbm, tail_hbm, rowtab_hbm, buf_v, tr_v):
    wid = lax.axis_index("s") * 2 + lax.axis_index("c")
    iota = lax.iota(jnp.int32, 16)
    base = [_W * (16 * k) + _W * iota for k in range(8)]

    def slab(vc, carry):
        @pl.when(vc < _NVC)
        def _():
            pltpu.sync_copy(tT_hbm.at[:, pl.ds(vc * 128, 128)], buf_v)

            def erow(e, c):
                for k in range(8):
                    vec = buf_v[e, pl.ds(16 * k, 16)]
                    off = base[k] + e
                    plsc.store_scatter(
                        tr_v,
                        [lax.shift_right_logical(off, 7),
                         lax.bitwise_and(off, 127)],
                        vec,
                    )
                return c

            lax.fori_loop(0, 64, erow, 0)
            pltpu.sync_copy(tr_v, rowtab_hbm.at[pl.ds(vc * _W, _W), :])

        return carry

    lax.fori_loop(0, (_NVC + _NW - 1) // _NW, lambda k, c: slab(wid + 32 * k, c), 0)

    @pl.when(wid == 4)
    def _tail():
        pltpu.sync_copy(tail_hbm, tr_v.at[pl.ds(0, _TAILR), :])
        pltpu.sync_copy(tr_v.at[pl.ds(0, _TAILR), :],
                        rowtab_hbm.at[pl.ds(_NVC * _W, _TAILR), :])


@functools.partial(
    pl.kernel,
    mesh=_mesh,
    out_type=jax.ShapeDtypeStruct((_HIST, 8, _BATCH // 128, 8, 128), jnp.float32),
    scratch_types=[
        pltpu.VMEM((128,), jnp.int32),
        pltpu.VMEM((128, _W), jnp.float32),
        pltpu.VMEM((8, 8, 128), jnp.float32),
        pltpu.SemaphoreType.DMA,
    ],
    compiler_params=pltpu.CompilerParams(
        use_tc_tiling_on_sc=False, needs_layout_passes=False),
)
def _gather(xT_hbm, rowtab_hbm, out_hbm, idx_v, rows_v, tr_v, sem):
    wid = lax.axis_index("s") * 2 + lax.axis_index("c")
    iota = lax.iota(jnp.int32, 16)
    rowsel = [16 * h + iota for h in range(8)]

    def unit(u, carry):
        t = u // 32
        bc = lax.rem(u, 32)
        pltpu.sync_copy(xT_hbm.at[t, pl.ds(bc * 128, 128)], idx_v)
        pltpu.async_copy(rowtab_hbm.at[idx_v], rows_v, sem).wait()

        for eb in range(8):
            for es in range(8):
                colv = jnp.full((16,), 0, jnp.int32) + (8 * eb + es)
                for h in range(8):
                    vec = plsc.load_gather(rows_v, [rowsel[h], colv])
                    tr_v[eb, es, pl.ds(16 * h, 16)] = vec

        pltpu.sync_copy(tr_v, out_hbm.at[t, :, bc])
        return carry

    lax.fori_loop(_UPW * wid, _UPW * (wid + 1), unit, 0)


def kernel(x, emb_table):
    tT = jnp.swapaxes(emb_table, 0, 1)
    xT = jnp.swapaxes(x, 0, 1).astype(jnp.int32)
    tail = (
        jnp.zeros((64, _W), jnp.float32)
        .at[:, :_EMB].set(emb_table[_NVC * 128:, :])
        .reshape(_TAILR, 128)
    )
    rowtab = _detile(tT, tail).reshape(_VOCAB, _W)
    out5 = _gather(xT, rowtab)
    return out5.transpose(2, 4, 0, 1, 3).reshape(_BATCH, _HIST, _EMB)


# speedup vs baseline: 2.1371x; 1.3463x over previous
import functools

import jax
import jax.numpy as jnp
from jax import lax
from jax.experimental import pallas as pl
from jax.experimental.pallas import tpu as pltpu
from jax.experimental.pallas import tpu_sc as plsc

_VOCAB = 1000000
_EMB = 64
_W = 80
_BATCH = 4096
_HIST = 200
_NW = 32
_NVC = _VOCAB // 128
_ROWS = _VOCAB * _W // 128
_TAILR = 64 * _W // 128
_KPW = (_NVC + _NW - 1) // _NW
_UNITS = _HIST * (_BATCH // 128)
_UPW = _UNITS // _NW

_mesh = plsc.VectorSubcoreMesh(core_axis_name="c", subcore_axis_name="s")


@functools.partial(
    pl.kernel,
    mesh=_mesh,
    out_type=jax.ShapeDtypeStruct((_ROWS, 128), jnp.float32),
    scratch_types=[
        pltpu.VMEM((64, 128), jnp.float32),
        pltpu.VMEM((64, 128), jnp.float32),
        pltpu.VMEM((_W, 128), jnp.float32),
        pltpu.VMEM((_W, 128), jnp.float32),
        pltpu.SemaphoreType.DMA,
        pltpu.SemaphoreType.DMA,
        pltpu.SemaphoreType.DMA,
        pltpu.SemaphoreType.DMA,
    ],
    compiler_params=pltpu.CompilerParams(
        use_tc_tiling_on_sc=True, needs_layout_passes=False),
)
def _detile(tT_hbm, tail_hbm, rowtab_hbm, buf0, buf1, tr0, tr1,
            si0, si1, so0, so1):
    wid = lax.axis_index("s") * 2 + lax.axis_index("c")
    iota = lax.iota(jnp.int32, 16)
    base = [_W * (16 * k) + _W * iota for k in range(8)]

    def vc_of(k):
        return wid + 32 * k

    def start_in(k, buf, sem):
        @pl.when(vc_of(k) < _NVC)
        def _():
            pltpu.async_copy(tT_hbm.at[:, pl.ds(vc_of(k) * 128, 128)], buf, sem)

    def wait_in(k, buf, sem):
        @pl.when(vc_of(k) < _NVC)
        def _():
            pltpu.make_async_copy(
                tT_hbm.at[:, pl.ds(vc_of(k) * 128, 128)], buf, sem).wait()

    def start_out(k, tr, sem):
        @pl.when(vc_of(k) < _NVC)
        def _():
            pltpu.async_copy(tr, rowtab_hbm.at[pl.ds(vc_of(k) * _W, _W), :], sem)

    def wait_out(k, tr, sem):
        @pl.when(vc_of(k) < _NVC)
        def _():
            pltpu.make_async_copy(
                tr, rowtab_hbm.at[pl.ds(vc_of(k) * _W, _W), :], sem).wait()

    def transpose(k, buf, tr):
        @pl.when(vc_of(k) < _NVC)
        def _():
            def erow(e, c):
                for j in range(8):
                    vec = buf[e, pl.ds(16 * j, 16)]
                    off = base[j] + e
                    plsc.store_scatter(
                        tr,
                        [lax.shift_right_logical(off, 7),
                         lax.bitwise_and(off, 127)],
                        vec,
                    )
                return c

            lax.fori_loop(0, 64, erow, 0)

    start_in(0, buf0, si0)
    start_in(1, buf1, si1)

    def step(p, carry):
        k0 = 2 * p
        wait_in(k0, buf0, si0)

        @pl.when(p > 0)
        def _():
            wait_out(k0 - 2, tr0, so0)

        transpose(k0, buf0, tr0)
        start_out(k0, tr0, so0)

        @pl.when(k0 + 2 < _KPW)
        def _():
            start_in(k0 + 2, buf0, si0)

        k1 = k0 + 1
        wait_in(k1, buf1, si1)

        @pl.when(p > 0)
        def _():
            wait_out(k1 - 2, tr1, so1)

        transpose(k1, buf1, tr1)
        start_out(k1, tr1, so1)

        @pl.when(k1 + 2 < _KPW)
        def _():
            start_in(k1 + 2, buf1, si1)

        return carry

    lax.fori_loop(0, _KPW // 2, step, 0)
    klast = _KPW - 1
    wait_in(klast, buf0, si0)
    wait_out(klast - 2, tr0, so0)
    transpose(klast, buf0, tr0)
    start_out(klast, tr0, so0)
    wait_out(klast - 1, tr1, so1)
    wait_out(klast, tr0, so0)

    @pl.when(wid == 4)
    def _tail():
        pltpu.sync_copy(tail_hbm, tr0.at[pl.ds(0, _TAILR), :])
        pltpu.sync_copy(tr0.at[pl.ds(0, _TAILR), :],
                        rowtab_hbm.at[pl.ds(_NVC * _W, _TAILR), :])


@functools.partial(
    pl.kernel,
    mesh=_mesh,
    out_type=jax.ShapeDtypeStruct((_HIST, 8, _BATCH // 128, 8, 128), jnp.float32),
    scratch_types=[
        pltpu.VMEM((128,), jnp.int32),
        pltpu.VMEM((128,), jnp.int32),
        pltpu.VMEM((128, _W), jnp.float32),
        pltpu.VMEM((128, _W), jnp.float32),
        pltpu.VMEM((8, 8, 128), jnp.float32),
        pltpu.VMEM((8, 8, 128), jnp.float32),
        pltpu.SemaphoreType.DMA,
        pltpu.SemaphoreType.DMA,
        pltpu.SemaphoreType.DMA,
        pltpu.SemaphoreType.DMA,
        pltpu.SemaphoreType.DMA,
        pltpu.SemaphoreType.DMA,
    ],
    compiler_params=pltpu.CompilerParams(
        use_tc_tiling_on_sc=False, needs_layout_passes=False),
)
def _gather(xT_hbm, rowtab_hbm, out_hbm, idx0, idx1, rows0, rows1, tra, trb,
            sg0, sg1, so0, so1, sx0, sx1):
    wid = lax.axis_index("s") * 2 + lax.axis_index("c")
    iota = lax.iota(jnp.int32, 16)
    rowsel = [16 * h + iota for h in range(8)]
    u_lo = _UPW * wid

    def tb(u):
        return u // 32, lax.rem(u, 32)

    def start_idx(u, idx, sem):
        t, bc = tb(u)
        pltpu.async_copy(xT_hbm.at[t, pl.ds(bc * 128, 128)], idx, sem)

    def wait_idx(u, idx, sem):
        t, bc = tb(u)
        pltpu.make_async_copy(xT_hbm.at[t, pl.ds(bc * 128, 128)], idx, sem).wait()

    def start_gather(idx, rows, sem):
        pltpu.async_copy(rowtab_hbm.at[idx], rows, sem)

    def wait_gather(idx, rows, sem):
        pltpu.make_async_copy(rowtab_hbm.at[idx], rows, sem).wait()

    def start_out(u, tr, sem):
        t, bc = tb(u)
        pltpu.async_copy(tr, out_hbm.at[t, :, bc], sem)

    def wait_out(u, tr, sem):
        t, bc = tb(u)
        pltpu.make_async_copy(tr, out_hbm.at[t, :, bc], sem).wait()

    def transpose(rows, tr):
        for eb in range(8):
            for es in range(8):
                colv = jnp.full((16,), 0, jnp.int32) + (8 * eb + es)
                for h in range(8):
                    vec = plsc.load_gather(rows, [rowsel[h], colv])
                    tr[eb, es, pl.ds(16 * h, 16)] = vec

    start_idx(u_lo + 0, idx0, sx0)
    start_idx(u_lo + 1, idx1, sx1)
    wait_idx(u_lo + 0, idx0, sx0)
    start_gather(idx0, rows0, sg0)
    wait_idx(u_lo + 1, idx1, sx1)
    start_gather(idx1, rows1, sg1)

    def step(p, carry):
        u0 = u_lo + 2 * p
        u1 = u0 + 1
        wait_gather(idx0, rows0, sg0)

        @pl.when(p + 1 < _UPW // 2)
        def _():
            start_idx(u0 + 2, idx0, sx0)

        @pl.when(p > 0)
        def _():
            wait_out(u0 - 2, tra, so0)

        transpose(rows0, tra)
        start_out(u0, tra, so0)

        @pl.when(p + 1 < _UPW // 2)
        def _():
            wait_idx(u0 + 2, idx0, sx0)
            start_gather(idx0, rows0, sg0)

        wait_gather(idx1, rows1, sg1)

        @pl.when(p + 1 < _UPW // 2)
        def _():
            start_idx(u1 + 2, idx1, sx1)

        @pl.when(p > 0)
        def _():
            wait_out(u1 - 2, trb, so1)

        transpose(rows1, trb)
        start_out(u1, trb, so1)

        @pl.when(p + 1 < _UPW // 2)
        def _():
            wait_idx(u1 + 2, idx1, sx1)
            start_gather(idx1, rows1, sg1)

        return carry

    lax.fori_loop(0, _UPW // 2, step, 0)
    wait_out(u_lo + _UPW - 2, tra, so0)
    wait_out(u_lo + _UPW - 1, trb, so1)


def kernel(x, emb_table):
    tT = jnp.swapaxes(emb_table, 0, 1)
    xT = jnp.swapaxes(x, 0, 1).astype(jnp.int32)
    tail = (
        jnp.zeros((64, _W), jnp.float32)
        .at[:, :_EMB].set(emb_table[_NVC * 128:, :])
        .reshape(_TAILR, 128)
    )
    rowtab = _detile(tT, tail).reshape(_VOCAB, _W)
    out5 = _gather(xT, rowtab)
    return out5.transpose(2, 4, 0, 1, 3).reshape(_BATCH, _HIST, _EMB)


# speedup vs baseline: 2.8354x; 1.3267x over previous
import functools

import jax
import jax.numpy as jnp
from jax import lax
from jax.experimental import pallas as pl
from jax.experimental.pallas import tpu as pltpu
from jax.experimental.pallas import tpu_sc as plsc

_VOCAB = 1000000
_EMB = 64
_BATCH = 4096
_HIST = 200
_B = _BATCH * _HIST
_NW = 32
_UNITS = _HIST * (_BATCH // 128)
_UPW = _UNITS // _NW

_mesh = plsc.VectorSubcoreMesh(core_axis_name="c", subcore_axis_name="s")


@functools.partial(
    pl.kernel,
    mesh=_mesh,
    out_type=jax.ShapeDtypeStruct((_B, _EMB), jnp.float32),
    scratch_types=[
        pltpu.VMEM((128,), jnp.int32),
        pltpu.VMEM((128,), jnp.int32),
        pltpu.VMEM((128,), jnp.int32),
        pltpu.VMEM((128,), jnp.int32),
        pltpu.VMEM((128, _EMB), jnp.float32),
        pltpu.VMEM((128, _EMB), jnp.float32),
        pltpu.VMEM((128, _EMB), jnp.float32),
        pltpu.VMEM((128, _EMB), jnp.float32),
        pltpu.SemaphoreType.DMA,
        pltpu.SemaphoreType.DMA,
        pltpu.SemaphoreType.DMA,
        pltpu.SemaphoreType.DMA,
        pltpu.SemaphoreType.DMA,
        pltpu.SemaphoreType.DMA,
        pltpu.SemaphoreType.DMA,
        pltpu.SemaphoreType.DMA,
        pltpu.SemaphoreType.DMA,
        pltpu.SemaphoreType.DMA,
        pltpu.SemaphoreType.DMA,
        pltpu.SemaphoreType.DMA,
    ],
    compiler_params=pltpu.CompilerParams(use_tc_tiling_on_sc=False),
)
def _emb_lookup(xT_hbm, table_hbm, out_hbm,
                ix0, ix1, ix2, ix3, rw0, rw1, rw2, rw3,
                sx0, sx1, sx2, sx3, sg0, sg1, sg2, sg3, so0, so1, so2, so3):
    wid = lax.axis_index("s") * 2 + lax.axis_index("c")
    u_lo = _UPW * wid
    idx = [ix0, ix1, ix2, ix3]
    rows = [rw0, rw1, rw2, rw3]
    sx = [sx0, sx1, sx2, sx3]
    sg = [sg0, sg1, sg2, sg3]
    so = [so0, so1, so2, so3]

    def tb(u):
        return u // 32, lax.rem(u, 32)

    def start_idx(u, s):
        t, bc = tb(u)
        pltpu.async_copy(xT_hbm.at[t, pl.ds(bc * 128, 128)], idx[s], sx[s])

    def wait_idx(u, s):
        t, bc = tb(u)
        pltpu.make_async_copy(
            xT_hbm.at[t, pl.ds(bc * 128, 128)], idx[s], sx[s]).wait()

    def start_gather(s):
        pltpu.async_copy(table_hbm.at[idx[s]], rows[s], sg[s])

    def wait_gather(s):
        pltpu.make_async_copy(table_hbm.at[idx[s]], rows[s], sg[s]).wait()

    def start_out(u, s):
        t, bc = tb(u)
        pltpu.async_copy(
            rows[s], out_hbm.at[pl.ds(t * _BATCH + bc * 128, 128), :], so[s])

    def wait_out(u, s):
        t, bc = tb(u)
        pltpu.make_async_copy(
            rows[s], out_hbm.at[pl.ds(t * _BATCH + bc * 128, 128), :],
            so[s]).wait()

    for s in range(4):
        start_idx(u_lo + s, s)
    wait_idx(u_lo + 0, 0)
    start_gather(0)
    wait_idx(u_lo + 1, 1)
    start_gather(1)

    def step(q, carry):
        for s in range(4):
            u = u_lo + 4 * q + s
            wait_gather(s)
            start_out(u, s)

            @pl.when(4 * q + s + 4 < _UPW)
            def _():
                start_idx(u + 4, s)

            nxt = (s + 2) % 4

            @pl.when(4 * q + s + 2 < _UPW)
            def _():
                @pl.when(4 * q + s >= 2)
                def _():
                    wait_out(u - 2, nxt)

                wait_idx(u + 2, nxt)
                start_gather(nxt)

        return carry

    lax.fori_loop(0, _UPW // 4, step, 0)
    wait_out(u_lo + _UPW - 4, 0)
    wait_out(u_lo + _UPW - 3, 1)
    wait_out(u_lo + _UPW - 2, 2)
    wait_out(u_lo + _UPW - 1, 3)


def kernel(x, emb_table):
    xT = jnp.swapaxes(x, 0, 1).astype(jnp.int32)
    outT = _emb_lookup(xT, emb_table)
    return outT.reshape(_HIST, _BATCH, _EMB).transpose(1, 0, 2)


# speedup vs baseline: 2.8398x; 1.0016x over previous
import functools

import jax
import jax.numpy as jnp
from jax import lax
from jax.experimental import pallas as pl
from jax.experimental.pallas import tpu as pltpu
from jax.experimental.pallas import tpu_sc as plsc

_VOCAB = 1000000
_EMB = 64
_BATCH = 4096
_HIST = 200
_B = _BATCH * _HIST
_NW = 32
_UNITS = _HIST * (_BATCH // 128)
_UPW = _UNITS // _NW

_mesh = plsc.VectorSubcoreMesh(core_axis_name="c", subcore_axis_name="s")


@functools.partial(
    pl.kernel,
    mesh=_mesh,
    out_type=jax.ShapeDtypeStruct((_B, _EMB), jnp.float32),
    scratch_types=[
        pltpu.VMEM((128,), jnp.int32),
        pltpu.VMEM((128,), jnp.int32),
        pltpu.VMEM((128,), jnp.int32),
        pltpu.VMEM((128,), jnp.int32),
        pltpu.VMEM((128, _EMB), jnp.float32),
        pltpu.VMEM((128, _EMB), jnp.float32),
        pltpu.VMEM((128, _EMB), jnp.float32),
        pltpu.VMEM((128, _EMB), jnp.float32),
        pltpu.SemaphoreType.DMA,
        pltpu.SemaphoreType.DMA,
        pltpu.SemaphoreType.DMA,
        pltpu.SemaphoreType.DMA,
        pltpu.SemaphoreType.DMA,
        pltpu.SemaphoreType.DMA,
        pltpu.SemaphoreType.DMA,
        pltpu.SemaphoreType.DMA,
        pltpu.SemaphoreType.DMA,
        pltpu.SemaphoreType.DMA,
        pltpu.SemaphoreType.DMA,
        pltpu.SemaphoreType.DMA,
    ],
    compiler_params=pltpu.CompilerParams(
        use_tc_tiling_on_sc=False, skip_device_barrier=True),
)
def _emb_lookup(xT_hbm, table_hbm, out_hbm,
                ix0, ix1, ix2, ix3, rw0, rw1, rw2, rw3,
                sx0, sx1, sx2, sx3, sg0, sg1, sg2, sg3, so0, so1, so2, so3):
    wid = lax.axis_index("s") * 2 + lax.axis_index("c")
    u_lo = _UPW * wid
    idx = [ix0, ix1, ix2, ix3]
    rows = [rw0, rw1, rw2, rw3]
    sx = [sx0, sx1, sx2, sx3]
    sg = [sg0, sg1, sg2, sg3]
    so = [so0, so1, so2, so3]

    def tb(u):
        return u // 32, lax.rem(u, 32)

    def start_idx(u, s):
        t, bc = tb(u)
        pltpu.async_copy(xT_hbm.at[t, pl.ds(bc * 128, 128)], idx[s], sx[s])

    def wait_idx(u, s):
        t, bc = tb(u)
        pltpu.make_async_copy(
            xT_hbm.at[t, pl.ds(bc * 128, 128)], idx[s], sx[s]).wait()

    def start_gather(s):
        pltpu.async_copy(table_hbm.at[idx[s]], rows[s], sg[s])

    def wait_gather(s):
        pltpu.make_async_copy(table_hbm.at[idx[s]], rows[s], sg[s]).wait()

    def start_out(u, s):
        t, bc = tb(u)
        pltpu.async_copy(
            rows[s], out_hbm.at[pl.ds(t * _BATCH + bc * 128, 128), :], so[s])

    def wait_out(u, s):
        t, bc = tb(u)
        pltpu.make_async_copy(
            rows[s], out_hbm.at[pl.ds(t * _BATCH + bc * 128, 128), :],
            so[s]).wait()

    for s in range(4):
        start_idx(u_lo + s, s)
    wait_idx(u_lo + 0, 0)
    start_gather(0)
    wait_idx(u_lo + 1, 1)
    start_gather(1)

    def step(q, carry):
        for s in range(4):
            u = u_lo + 4 * q + s
            wait_gather(s)
            start_out(u, s)

            @pl.when(4 * q + s + 4 < _UPW)
            def _():
                start_idx(u + 4, s)

            nxt = (s + 2) % 4

            @pl.when(4 * q + s + 2 < _UPW)
            def _():
                @pl.when(4 * q + s >= 2)
                def _():
                    wait_out(u - 2, nxt)

                wait_idx(u + 2, nxt)
                start_gather(nxt)

        return carry

    lax.fori_loop(0, _UPW // 4, step, 0)
    wait_out(u_lo + _UPW - 4, 0)
    wait_out(u_lo + _UPW - 3, 1)
    wait_out(u_lo + _UPW - 2, 2)
    wait_out(u_lo + _UPW - 1, 3)


def kernel(x, emb_table):
    xT = jnp.swapaxes(x, 0, 1).astype(jnp.int32)
    outT = _emb_lookup(xT, emb_table)
    return outT.reshape(_HIST, _BATCH, _EMB).transpose(1, 0, 2)


# speedup vs baseline: 2.8617x; 1.0077x over previous
import functools

import jax
import jax.numpy as jnp
from jax import lax
from jax.experimental import pallas as pl
from jax.experimental.pallas import tpu as pltpu
from jax.experimental.pallas import tpu_sc as plsc

_VOCAB = 1000000
_EMB = 64
_BATCH = 4096
_HIST = 200
_B = _BATCH * _HIST
_NW = 32
_UNITS = _HIST * (_BATCH // 256)
_UPW = _UNITS // _NW

_mesh = plsc.VectorSubcoreMesh(core_axis_name="c", subcore_axis_name="s")


@functools.partial(
    pl.kernel,
    mesh=_mesh,
    out_type=jax.ShapeDtypeStruct((_B, _EMB), jnp.float32),
    scratch_types=[
        pltpu.VMEM((256,), jnp.int32),
        pltpu.VMEM((256,), jnp.int32),
        pltpu.VMEM((256,), jnp.int32),
        pltpu.VMEM((256,), jnp.int32),
        pltpu.VMEM((256, _EMB), jnp.float32),
        pltpu.VMEM((256, _EMB), jnp.float32),
        pltpu.VMEM((256, _EMB), jnp.float32),
        pltpu.VMEM((256, _EMB), jnp.float32),
        pltpu.SemaphoreType.DMA,
        pltpu.SemaphoreType.DMA,
        pltpu.SemaphoreType.DMA,
        pltpu.SemaphoreType.DMA,
        pltpu.SemaphoreType.DMA,
        pltpu.SemaphoreType.DMA,
        pltpu.SemaphoreType.DMA,
        pltpu.SemaphoreType.DMA,
        pltpu.SemaphoreType.DMA,
        pltpu.SemaphoreType.DMA,
        pltpu.SemaphoreType.DMA,
        pltpu.SemaphoreType.DMA,
    ],
    compiler_params=pltpu.CompilerParams(
        use_tc_tiling_on_sc=False, skip_device_barrier=True),
)
def _emb_lookup(xT_hbm, table_hbm, out_hbm,
                ix0, ix1, ix2, ix3, rw0, rw1, rw2, rw3,
                sx0, sx1, sx2, sx3, sg0, sg1, sg2, sg3, so0, so1, so2, so3):
    wid = lax.axis_index("s") * 2 + lax.axis_index("c")
    u_lo = _UPW * wid
    idx = [ix0, ix1, ix2, ix3]
    rows = [rw0, rw1, rw2, rw3]
    sx = [sx0, sx1, sx2, sx3]
    sg = [sg0, sg1, sg2, sg3]
    so = [so0, so1, so2, so3]

    def tb(u):
        return u // 16, lax.rem(u, 16)

    def start_idx(u, s):
        t, bc = tb(u)
        pltpu.async_copy(xT_hbm.at[t, pl.ds(bc * 256, 256)], idx[s], sx[s])

    def wait_idx(u, s):
        t, bc = tb(u)
        pltpu.make_async_copy(
            xT_hbm.at[t, pl.ds(bc * 256, 256)], idx[s], sx[s]).wait()

    def start_gather(s):
        pltpu.async_copy(table_hbm.at[idx[s]], rows[s], sg[s])

    def wait_gather(s):
        pltpu.make_async_copy(table_hbm.at[idx[s]], rows[s], sg[s]).wait()

    def start_out(u, s):
        t, bc = tb(u)
        pltpu.async_copy(
            rows[s], out_hbm.at[pl.ds(t * _BATCH + bc * 256, 256), :], so[s])

    def wait_out(u, s):
        t, bc = tb(u)
        pltpu.make_async_copy(
            rows[s], out_hbm.at[pl.ds(t * _BATCH + bc * 256, 256), :],
            so[s]).wait()

    for s in range(4):
        start_idx(u_lo + s, s)
    wait_idx(u_lo + 0, 0)
    start_gather(0)
    wait_idx(u_lo + 1, 1)
    start_gather(1)

    def step(q, carry):
        for s in range(4):
            u = u_lo + 4 * q + s
            wait_gather(s)
            start_out(u, s)

            @pl.when(4 * q + s + 4 < _UPW)
            def _():
                start_idx(u + 4, s)

            nxt = (s + 2) % 4

            @pl.when(4 * q + s + 2 < _UPW)
            def _():
                @pl.when(4 * q + s >= 2)
                def _():
                    wait_out(u - 2, nxt)

                wait_idx(u + 2, nxt)
                start_gather(nxt)

        return carry

    lax.fori_loop(0, _UPW // 4, step, 0)
    wait_out(u_lo + _UPW - 4, 0)
    wait_out(u_lo + _UPW - 3, 1)
    wait_out(u_lo + _UPW - 2, 2)
    wait_out(u_lo + _UPW - 1, 3)


def kernel(x, emb_table):
    xT = jnp.swapaxes(x, 0, 1).astype(jnp.int32)
    outT = _emb_lookup(xT, emb_table)
    return outT.reshape(_HIST, _BATCH, _EMB).transpose(1, 0, 2)
